# Initial kernel scaffold; baseline (speedup 1.0000x reference)
import jax
import jax.numpy as jnp
from jax.experimental import pallas as pl


def kernel(input, table0, table1):
    raise NotImplementedError("write your pallas kernel here")


# speedup vs baseline: 1.7561x; 1.7561x over previous
import functools

import jax
import jax.numpy as jnp
from jax import lax
from jax.experimental import pallas as pl
from jax.experimental.pallas import tpu as pltpu
from jax.experimental.pallas import tpu_sc as plsc

DIM = 32
NUM_WORKERS = 32
CHUNK = 1024


@functools.partial(jax.jit, static_argnums=(3, 4))
def _combine(idx_flat, table0, table1, total, per_worker):
    n_chunks = per_worker // CHUNK
    mesh = plsc.VectorSubcoreMesh(core_axis_name="c", subcore_axis_name="s")

    @functools.partial(
        pl.kernel,
        mesh=mesh,
        compiler_params=pltpu.CompilerParams(use_tc_tiling_on_sc=False),
        out_type=jax.ShapeDtypeStruct((total, 2 * DIM), jnp.float32),
        scratch_types=[
            pltpu.VMEM((CHUNK,), jnp.int32),
            pltpu.VMEM((CHUNK, DIM), jnp.float32),
            pltpu.VMEM((CHUNK, DIM), jnp.float32),
            pltpu.SemaphoreType.DMA,
            pltpu.SemaphoreType.DMA,
        ],
    )
    def k(idx_hbm, t0_hbm, t1_hbm, out_hbm, idx_v, r0_v, r1_v, sem0, sem1):
        wid = lax.axis_index("s") * 2 + lax.axis_index("c")
        base_w = wid * per_worker

        def body(i, carry):
            base = base_w + i * CHUNK
            pltpu.sync_copy(idx_hbm.at[pl.ds(base, CHUNK)], idx_v)
            c0 = pltpu.async_copy(t0_hbm.at[idx_v], r0_v, sem0)
            c1 = pltpu.async_copy(t1_hbm.at[idx_v], r1_v, sem1)
            c0.wait()
            c1.wait()
            pltpu.sync_copy(r0_v, out_hbm.at[pl.ds(base, CHUNK), pl.ds(0, DIM)])
            pltpu.sync_copy(r1_v, out_hbm.at[pl.ds(base, CHUNK), pl.ds(DIM, DIM)])
            return carry

        lax.fori_loop(0, n_chunks, body, 0)

    return k(idx_flat, table0, table1)


def kernel(input, table0, table1):
    B, L = input.shape
    total = B * L
    idx_flat = input.reshape(total).astype(jnp.int32)
    per_worker = total // NUM_WORKERS
    out = _combine(idx_flat, table0, table1, total, per_worker)
    return out.reshape(B, L, 2 * DIM)


# speedup vs baseline: 1.7701x; 1.0080x over previous
import functools

import jax
import jax.numpy as jnp
from jax import lax
from jax.experimental import pallas as pl
from jax.experimental.pallas import tpu as pltpu
from jax.experimental.pallas import tpu_sc as plsc

DIM = 32
NUM_WORKERS = 32
CHUNK = 512
NBUF = 2


@functools.partial(jax.jit, static_argnums=(3, 4))
def _combine(idx_flat, table0, table1, total, per_worker):
    n_chunks = per_worker // CHUNK
    assert per_worker % CHUNK == 0 and n_chunks % NBUF == 0
    mesh = plsc.VectorSubcoreMesh(core_axis_name="c", subcore_axis_name="s")

    row_bufs = [
        [pltpu.VMEM((CHUNK, DIM), jnp.float32) for _ in range(2)]
        for _ in range(NBUF)
    ]
    gather_sems = [pltpu.SemaphoreType.DMA for _ in range(NBUF)]
    write_sems = [pltpu.SemaphoreType.DMA for _ in range(NBUF)]

    @functools.partial(
        pl.kernel,
        mesh=mesh,
        compiler_params=pltpu.CompilerParams(use_tc_tiling_on_sc=False),
        out_type=jax.ShapeDtypeStruct((total, 2 * DIM), jnp.float32),
        scratch_types=[pltpu.VMEM((per_worker,), jnp.int32), row_bufs,
                       gather_sems, write_sems],
    )
    def k(idx_hbm, t0_hbm, t1_hbm, out_hbm, idx_v, rbufs, gsems, wsems):
        wid = lax.axis_index("s") * 2 + lax.axis_index("c")
        base_w = wid * per_worker
        pltpu.sync_copy(idx_hbm.at[pl.ds(base_w, per_worker)], idx_v)

        def fire_gathers(i, b):
            sl = idx_v.at[pl.ds(i * CHUNK, CHUNK)]
            pltpu.async_copy(t0_hbm.at[sl], rbufs[b][0], gsems[b])
            pltpu.async_copy(t1_hbm.at[sl], rbufs[b][1], gsems[b])

        def wait_gathers(i, b):
            pltpu.make_async_copy(t0_hbm.at[idx_v.at[pl.ds(0, CHUNK)]],
                                  rbufs[b][0], gsems[b]).wait()
            pltpu.make_async_copy(t1_hbm.at[idx_v.at[pl.ds(0, CHUNK)]],
                                  rbufs[b][1], gsems[b]).wait()

        def fire_writes(i, b):
            base = base_w + i * CHUNK
            pltpu.async_copy(rbufs[b][0],
                             out_hbm.at[pl.ds(base, CHUNK), pl.ds(0, DIM)],
                             wsems[b])
            pltpu.async_copy(rbufs[b][1],
                             out_hbm.at[pl.ds(base, CHUNK), pl.ds(DIM, DIM)],
                             wsems[b])

        def wait_writes(b):
            pltpu.make_async_copy(rbufs[b][0],
                                  out_hbm.at[pl.ds(0, CHUNK), pl.ds(0, DIM)],
                                  wsems[b]).wait()
            pltpu.make_async_copy(rbufs[b][1],
                                  out_hbm.at[pl.ds(0, CHUNK), pl.ds(DIM, DIM)],
                                  wsems[b]).wait()

        for b in range(NBUF):
            fire_gathers(b, b)

        def body(g, carry):
            for b in range(NBUF):
                i = g + b
                wait_gathers(i, b)
                fire_writes(i, b)
                wait_writes(b)
                fire_gathers(i + NBUF, b)
            return carry

        lax.fori_loop(0, (n_chunks - NBUF) // NBUF, lambda t, c: body(t * NBUF, c),
                      0, unroll=False)

        g0 = n_chunks - NBUF
        for b in range(NBUF):
            i = g0 + b
            wait_gathers(i, b)
            fire_writes(i, b)
            wait_writes(b)

    return k(idx_flat, table0, table1)


def kernel(input, table0, table1):
    B, L = input.shape
    total = B * L
    idx_flat = input.reshape(total).astype(jnp.int32)
    per_worker = total // NUM_WORKERS
    out = _combine(idx_flat, table0, table1, total, per_worker)
    return out.reshape(B, L, 2 * DIM)


# speedup vs baseline: 1.8229x; 1.0298x over previous
import functools

import jax
import jax.numpy as jnp
from jax import lax
from jax.experimental import pallas as pl
from jax.experimental.pallas import tpu as pltpu
from jax.experimental.pallas import tpu_sc as plsc

DIM = 32
NUM_WORKERS = 32
CHUNK = 512
NBUF = 2


@functools.partial(jax.jit, static_argnums=(3, 4))
def _combine(idx_flat, table0, table1, total, per_worker):
    n_chunks = per_worker // CHUNK
    assert per_worker % CHUNK == 0 and n_chunks % NBUF == 0
    mesh = plsc.VectorSubcoreMesh(core_axis_name="c", subcore_axis_name="s")

    row_bufs = [
        [pltpu.VMEM((CHUNK, DIM), jnp.float32) for _ in range(2)]
        for _ in range(NBUF)
    ]
    gather_sems = [pltpu.SemaphoreType.DMA for _ in range(NBUF)]
    write_sems = [pltpu.SemaphoreType.DMA for _ in range(NBUF)]

    @functools.partial(
        pl.kernel,
        mesh=mesh,
        compiler_params=pltpu.CompilerParams(use_tc_tiling_on_sc=False),
        out_type=jax.ShapeDtypeStruct((total, 2 * DIM), jnp.float32),
        scratch_types=[pltpu.VMEM((per_worker,), jnp.int32), row_bufs,
                       gather_sems, write_sems],
    )
    def k(idx_hbm, t0_hbm, t1_hbm, out_hbm, idx_v, rbufs, gsems, wsems):
        wid = lax.axis_index("s") * 2 + lax.axis_index("c")
        base_w = wid * per_worker
        pltpu.sync_copy(idx_hbm.at[pl.ds(base_w, per_worker)], idx_v)

        def fire_gathers(i, b):
            sl = idx_v.at[pl.ds(i * CHUNK, CHUNK)]
            pltpu.async_copy(t0_hbm.at[sl], rbufs[b][0], gsems[b])
            pltpu.async_copy(t1_hbm.at[sl], rbufs[b][1], gsems[b])

        def wait_gathers(i, b):
            pltpu.make_async_copy(t0_hbm.at[idx_v.at[pl.ds(0, CHUNK)]],
                                  rbufs[b][0], gsems[b]).wait()
            pltpu.make_async_copy(t1_hbm.at[idx_v.at[pl.ds(0, CHUNK)]],
                                  rbufs[b][1], gsems[b]).wait()

        def fire_writes(i, b):
            base = base_w + i * CHUNK
            pltpu.async_copy(rbufs[b][0],
                             out_hbm.at[pl.ds(base, CHUNK), pl.ds(0, DIM)],
                             wsems[b])
            pltpu.async_copy(rbufs[b][1],
                             out_hbm.at[pl.ds(base, CHUNK), pl.ds(DIM, DIM)],
                             wsems[b])

        def wait_writes(b):
            pltpu.make_async_copy(rbufs[b][0],
                                  out_hbm.at[pl.ds(0, CHUNK), pl.ds(0, DIM)],
                                  wsems[b]).wait()
            pltpu.make_async_copy(rbufs[b][1],
                                  out_hbm.at[pl.ds(0, CHUNK), pl.ds(DIM, DIM)],
                                  wsems[b]).wait()

        for b in range(NBUF):
            fire_gathers(b, b)

        def body(g, carry):
            for b in range(NBUF):
                i = g + b
                wait_gathers(i, b)
                fire_writes(i, b)
                wait_writes(b)
                fire_gathers(i + NBUF, b)
            return carry

        lax.fori_loop(0, (n_chunks - NBUF) // NBUF, lambda t, c: body(t * NBUF, c),
                      0, unroll=False)

        g0 = n_chunks - NBUF
        for b in range(NBUF):
            i = g0 + b
            wait_gathers(i, b)
            fire_writes(i, b)
            wait_writes(b)

    return k(idx_flat, table0, table1)


def kernel(input, table0, table1):
    B, L = input.shape
    total = B * L
    idx_flat = input.T.reshape(total).astype(jnp.int32)
    per_worker = total // NUM_WORKERS
    out = _combine(idx_flat, table0, table1, total, per_worker)
    return out.reshape(L, B, 2 * DIM).transpose(1, 0, 2)
